# baseline (speedup 1.0000x reference)
import jax
import jax.numpy as jnp
from jax.experimental import pallas as pl
from jax.experimental.pallas import tpu as pltpu

_CHUNKS = 16
_NBUF = 16


def _pipe_body(q_hbm, o_hbm, buf, insem, outsem):
    cols = q_hbm.shape[1]
    ch = cols // _CHUNKS

    def in_copy(i):
        return pltpu.make_async_copy(
            q_hbm.at[:, pl.ds(i * ch, ch)], buf.at[i % _NBUF],
            insem.at[i % _NBUF])

    def out_copy(i):
        return pltpu.make_async_copy(
            buf.at[i % _NBUF], o_hbm.at[:, pl.ds(i * ch, ch)],
            outsem.at[i % _NBUF])

    for i in range(_NBUF):
        in_copy(i).start()
    for i in range(_CHUNKS):
        in_copy(i).wait()
        out_copy(i).start()
        j = i + _NBUF
        if j < _CHUNKS:
            out_copy(i).wait()
            in_copy(j).start()
    for i in range(_CHUNKS - _NBUF, _CHUNKS):
        out_copy(i).wait()


def kernel(c, q, emb_table):
    del c, emb_table
    qt = q.T
    cols, rows = qt.shape
    ch = rows // _CHUNKS
    out_t = pl.pallas_call(
        _pipe_body,
        in_specs=[pl.BlockSpec(memory_space=pl.ANY)],
        out_specs=pl.BlockSpec(memory_space=pl.ANY),
        out_shape=jax.ShapeDtypeStruct((cols, rows), q.dtype),
        scratch_shapes=[
            pltpu.VMEM((_NBUF, cols, ch), jnp.float32),
            pltpu.SemaphoreType.DMA((_NBUF,)),
            pltpu.SemaphoreType.DMA((_NBUF,)),
        ],
    )(qt)
    return out_t.T


# speedup vs baseline: 1.0694x; 1.0694x over previous
import jax
import jax.numpy as jnp
from jax.experimental import pallas as pl
from jax.experimental.pallas import tpu as pltpu

_CHUNKS = 4
_NBUF = 4


def _pipe_body(q_hbm, o_hbm, buf, insem, outsem):
    cols = q_hbm.shape[1]
    ch = cols // _CHUNKS

    def in_copy(i):
        return pltpu.make_async_copy(
            q_hbm.at[:, pl.ds(i * ch, ch)], buf.at[i % _NBUF],
            insem.at[i % _NBUF])

    def out_copy(i):
        return pltpu.make_async_copy(
            buf.at[i % _NBUF], o_hbm.at[:, pl.ds(i * ch, ch)],
            outsem.at[i % _NBUF])

    for i in range(_NBUF):
        in_copy(i).start()
    for i in range(_CHUNKS):
        in_copy(i).wait()
        out_copy(i).start()
        j = i + _NBUF
        if j < _CHUNKS:
            out_copy(i).wait()
            in_copy(j).start()
    for i in range(_CHUNKS - _NBUF, _CHUNKS):
        out_copy(i).wait()


def kernel(c, q, emb_table):
    del c, emb_table
    qt = q.T
    cols, rows = qt.shape
    ch = rows // _CHUNKS
    out_t = pl.pallas_call(
        _pipe_body,
        in_specs=[pl.BlockSpec(memory_space=pl.ANY)],
        out_specs=pl.BlockSpec(memory_space=pl.ANY),
        out_shape=jax.ShapeDtypeStruct((cols, rows), q.dtype),
        scratch_shapes=[
            pltpu.VMEM((_NBUF, cols, ch), jnp.float32),
            pltpu.SemaphoreType.DMA((_NBUF,)),
            pltpu.SemaphoreType.DMA((_NBUF,)),
        ],
    )(qt)
    return out_t.T
